# baseline (speedup 1.0000x reference)
import functools

import jax
import jax.numpy as jnp
from jax import lax
from jax.experimental import pallas as pl
from jax.experimental.pallas import tpu as pltpu
from jax.experimental.pallas import tpu_sc as plsc

N, E, D, DE, H = 10000, 320000, 128, 16, 64
AGG = 32.0

NC, NS = 2, 16
NW = NC * NS
NSEG = 5
ESEG = E // NSEG
EPW = ESEG // NW
K = 80
NCHUNK = EPW // K
NP = 10240
ROWS_PER_TILE = NP // NS
ZROWS = 32

_BN = 2000
_BE = 8000


def _silu(a):
    return 0.5 * a * (1.0 + jnp.tanh(0.5 * a))


def _pack2(w):
    rnd = jnp.uint32(0x8000)
    top = jax.lax.bitcast_convert_type(w[:, :D // 2], jnp.uint32) + rnd
    low = jax.lax.bitcast_convert_type(w[:, D // 2:], jnp.uint32) + rnd
    return (top & jnp.uint32(0xFFFF0000)) | (low >> jnp.uint32(16))


def _xup_body(nf_ref, w_ref, o_ref):
    o_ref[...] = jnp.dot(nf_ref[...], w_ref[...],
                         preferred_element_type=jnp.float32) * (D ** -0.5)


def _bdot(a, b):
    return jnp.dot(a.astype(jnp.bfloat16), b.astype(jnp.bfloat16),
                   preferred_element_type=jnp.float32)


def _mlp_body(ef_ref, ea_ref, w1_ref, w2_ref, w3_ref, w4_ref, o_ref):
    h = _silu(_bdot(ef_ref[...], w1_ref[...]) * (DE ** -0.5))
    h = _silu(_bdot(h, w2_ref[...]) * (H ** -0.5))
    h = _silu(_bdot(h, w3_ref[...]) * (H ** -0.5))
    w = _bdot(h, w4_ref[...]) * (H ** -0.5)
    o_ref[...] = _pack2(w * ea_ref[...])


def _final_body(*refs):
    acc_refs, wl_ref, o_ref = refs[:-2], refs[-2], refs[-1]
    m = acc_refs[0][0] + acc_refs[0][1]
    for a in acc_refs[1:]:
        m = m + a[0] + a[1]
    o_ref[...] = jnp.dot(m, wl_ref[...],
                         preferred_element_type=jnp.float32) * (1.0 / (AGG * D ** 0.5))


def _sc_body(x_hbm, tpw_hbm, sidx_hbm, ridx_hbm, out_hbm,
             sidx0, ridx0, tpw0, xr0, sidx1, ridx1, tpw1, xr1,
             zbuf_v, acc_sh,
             semA0, semA1, semG0, semG1, semS0, semS1):
    c = lax.axis_index("c")
    s = lax.axis_index("s")
    base = (c * NS + s) * EPW

    bufs = ((sidx0, ridx0, tpw0, xr0, semA0, semG0, semS0),
            (sidx1, ridx1, tpw1, xr1, semA1, semG1, semS1))

    def _zero_row(r, _):
        for j in range(D // 16):
            zbuf_v[r, pl.ds(j * 16, 16)] = jnp.zeros((16,), jnp.float32)
        return 0
    lax.fori_loop(0, ZROWS, _zero_row, 0)
    for z in range(ROWS_PER_TILE // ZROWS):
        pltpu.sync_copy(zbuf_v, acc_sh.at[pl.ds(s * ROWS_PER_TILE + z * ZROWS, ZROWS)])
    plsc.subcore_barrier()

    def _startA(i, p):
        off = base + i * K
        sidx, ridx, tpw, semA = bufs[p][0], bufs[p][1], bufs[p][2], bufs[p][4]
        pltpu.async_copy(sidx_hbm.at[pl.ds(off, K)], sidx, semA)
        pltpu.async_copy(ridx_hbm.at[pl.ds(off, K)], ridx, semA)
        pltpu.async_copy(tpw_hbm.at[pl.ds(off, K)], tpw, semA)

    def _waitA(i, p):
        off = base + i * K
        sidx, ridx, tpw, semA = bufs[p][0], bufs[p][1], bufs[p][2], bufs[p][4]
        pltpu.make_async_copy(sidx_hbm.at[pl.ds(off, K)], sidx, semA).wait()
        pltpu.make_async_copy(ridx_hbm.at[pl.ds(off, K)], ridx, semA).wait()
        pltpu.make_async_copy(tpw_hbm.at[pl.ds(off, K)], tpw, semA).wait()

    def _gather(p):
        return pltpu.async_copy(x_hbm.at[bufs[p][0]], bufs[p][3], bufs[p][5])

    def _startS(p):
        pltpu.async_copy(bufs[p][3], acc_sh.at[bufs[p][1]], bufs[p][6], add=True)

    def _waitS(p):
        pltpu.make_async_copy(bufs[p][3], acc_sh.at[bufs[p][1]], bufs[p][6]).wait()

    _MSK = jnp.uint32(0xFFFF0000)
    _S16 = jnp.uint32(16)

    def _mul(p):
        tpw, xr = bufs[p][2], bufs[p][3]

        def _m(e, _):
            for q in range(D // 32):
                ut = tpw[e, pl.ds(q * 16, 16)]
                ft = jax.lax.bitcast_convert_type(ut & _MSK, jnp.float32)
                sl1 = pl.ds(q * 16, 16)
                xr[e, sl1] = ft * xr[e, sl1]
                gt = jax.lax.bitcast_convert_type(ut << _S16, jnp.float32)
                sl2 = pl.ds(D // 2 + q * 16, 16)
                xr[e, sl2] = gt * xr[e, sl2]
            return 0
        lax.fori_loop(0, K, _m, 0)

    _startA(0, 0)

    def _body(t, _):
        i0 = 2 * t
        _waitA(i0, 0)
        g0 = _gather(0)

        @pl.when(t > 0)
        def _():
            _waitS(1)
        _startA(i0 + 1, 1)
        g0.wait()
        _mul(0)
        _startS(0)

        _waitA(i0 + 1, 1)
        g1 = _gather(1)
        _waitS(0)
        _startA(i0 + 2, 0)
        g1.wait()
        _mul(1)
        _startS(1)
        return 0
    lax.fori_loop(0, (NCHUNK - 1) // 2, _body, 0)

    _waitA(NCHUNK - 1, 0)
    gL = _gather(0)
    _waitS(1)
    gL.wait()
    _mul(0)
    _startS(0)
    _waitS(0)

    plsc.subcore_barrier()
    r0 = s * ROWS_PER_TILE
    pltpu.sync_copy(acc_sh.at[pl.ds(r0, ROWS_PER_TILE)],
                    out_hbm.at[c, pl.ds(r0, ROWS_PER_TILE)])


_sc_scatter = pl.kernel(
    _sc_body,
    mesh=plsc.VectorSubcoreMesh(core_axis_name="c", subcore_axis_name="s"),
    out_type=jax.ShapeDtypeStruct((NC, NP, D), jnp.float32),
    scratch_types=[
        pltpu.VMEM((K,), jnp.int32),
        pltpu.VMEM((K,), jnp.int32),
        pltpu.VMEM((K, D // 2), jnp.uint32),
        pltpu.VMEM((K, D), jnp.float32),
        pltpu.VMEM((K,), jnp.int32),
        pltpu.VMEM((K,), jnp.int32),
        pltpu.VMEM((K, D // 2), jnp.uint32),
        pltpu.VMEM((K, D), jnp.float32),
        pltpu.VMEM((ZROWS, D), jnp.float32),
        pltpu.VMEM_SHARED((NP, D), jnp.float32),
        pltpu.SemaphoreType.DMA,
        pltpu.SemaphoreType.DMA,
        pltpu.SemaphoreType.DMA,
        pltpu.SemaphoreType.DMA,
        pltpu.SemaphoreType.DMA,
        pltpu.SemaphoreType.DMA,
    ],
)


def kernel(node_feats, edge_attrs, edge_feats, edge_index, W_up, W1, W2, W3, W4, W_lin):
    sender = edge_index[0]
    receiver = edge_index[1]

    x = pl.pallas_call(
        _xup_body,
        out_shape=jax.ShapeDtypeStruct((N, D), jnp.float32),
        grid=(N // _BN,),
        in_specs=[
            pl.BlockSpec((_BN, D), lambda i: (i, 0)),
            pl.BlockSpec((D, D), lambda i: (0, 0)),
        ],
        out_specs=pl.BlockSpec((_BN, D), lambda i: (i, 0)),
    )(node_feats, W_up)

    accs = []
    for seg in range(NSEG):
        lo = seg * ESEG
        tpw2_s = pl.pallas_call(
            _mlp_body,
            out_shape=jax.ShapeDtypeStruct((ESEG, D // 2), jnp.uint32),
            grid=(ESEG // _BE,),
            in_specs=[
                pl.BlockSpec((_BE, DE), lambda i: (i, 0)),
                pl.BlockSpec((_BE, 1), lambda i: (i, 0)),
                pl.BlockSpec((DE, H), lambda i: (0, 0)),
                pl.BlockSpec((H, H), lambda i: (0, 0)),
                pl.BlockSpec((H, H), lambda i: (0, 0)),
                pl.BlockSpec((H, D), lambda i: (0, 0)),
            ],
            out_specs=pl.BlockSpec((_BE, D // 2), lambda i: (i, 0)),
        )(edge_feats[lo:lo + ESEG], edge_attrs[lo:lo + ESEG], W1, W2, W3, W4)

        accs.append(_sc_scatter(x, tpw2_s,
                                sender[lo:lo + ESEG], receiver[lo:lo + ESEG]))

    out = pl.pallas_call(
        _final_body,
        out_shape=jax.ShapeDtypeStruct((N, D), jnp.float32),
        grid=(N // _BN,),
        in_specs=[pl.BlockSpec((NC, _BN, D), lambda i: (0, i, 0))] * NSEG + [
            pl.BlockSpec((D, D), lambda i: (0, 0)),
        ],
        out_specs=pl.BlockSpec((_BN, D), lambda i: (i, 0)),
    )(*accs, W_lin)

    return out.reshape(N, D, 1)


# speedup vs baseline: 1.0582x; 1.0582x over previous
import functools

import jax
import jax.numpy as jnp
from jax import lax
from jax.experimental import pallas as pl
from jax.experimental.pallas import tpu as pltpu
from jax.experimental.pallas import tpu_sc as plsc

N, E, D, DE, H = 10000, 320000, 128, 16, 64
AGG = 32.0

NC, NS = 2, 16
NW = NC * NS
NSEG = 5
ESEG = E // NSEG
EPW = ESEG // NW
K = 80
NCHUNK = EPW // K
NP = 10240
ROWS_PER_TILE = NP // NS
ZROWS = 32

_BN = 2000
_BE = 8000


def _silu(a):
    return 0.5 * a * (1.0 + jnp.tanh(0.5 * a))


def _pack2(w):
    msk = jnp.uint32(0xFFFF0000)
    u = jax.lax.bitcast_convert_type(w, jnp.uint32) + jnp.uint32(0x8000)
    g = u.reshape(u.shape[0] // K, 2, K // 2, D)
    ua, ub = g[:, 0], g[:, 1]
    wa = (ua[..., :D // 2] & msk) | (ua[..., D // 2:] >> jnp.uint32(16))
    wb = (ub[..., :D // 2] & msk) | (ub[..., D // 2:] >> jnp.uint32(16))
    return jnp.concatenate([wa, wb], axis=-1).reshape(u.shape[0] // 2, D)


def _xup_body(nf_ref, w_ref, o_ref):
    o_ref[...] = jnp.dot(nf_ref[...], w_ref[...],
                         preferred_element_type=jnp.float32) * (D ** -0.5)


def _bdot(a, b):
    return jnp.dot(a.astype(jnp.bfloat16), b.astype(jnp.bfloat16),
                   preferred_element_type=jnp.float32)


def _mlp_body(ef_ref, ea_ref, w1_ref, w2_ref, w3_ref, w4_ref, o_ref):
    h = _silu(_bdot(ef_ref[...], w1_ref[...]) * (DE ** -0.5))
    h = _silu(_bdot(h, w2_ref[...]) * (H ** -0.5))
    h = _silu(_bdot(h, w3_ref[...]) * (H ** -0.5))
    w = _bdot(h, w4_ref[...]) * (H ** -0.5)
    o_ref[...] = _pack2(w * ea_ref[...])


def _final_body(*refs):
    acc_refs, wl_ref, o_ref = refs[:-2], refs[-2], refs[-1]
    m = acc_refs[0][0] + acc_refs[0][1]
    for a in acc_refs[1:]:
        m = m + a[0] + a[1]
    o_ref[...] = jnp.dot(m, wl_ref[...],
                         preferred_element_type=jnp.float32) * (1.0 / (AGG * D ** 0.5))


def _sc_body(x_hbm, tpw_hbm, sidx_hbm, ridx_hbm, out_hbm,
             sidx0, ridx0, tpw0, sidx1, ridx1, tpw1,
             sidx2, ridx2, tpw2, sidx3, ridx3, tpw3,
             xr0, xr1, zbuf_v, acc_sh,
             semA0, semA1, semA2, semA3, semG0, semG1, semS0, semS1):
    c = lax.axis_index("c")
    s = lax.axis_index("s")
    base = (c * NS + s) * EPW

    sidx = (sidx0, sidx1, sidx2, sidx3)
    ridx = (ridx0, ridx1, ridx2, ridx3)
    tpwb = (tpw0, tpw1, tpw2, tpw3)
    semA = (semA0, semA1, semA2, semA3)
    xr = (xr0, xr1)
    semG = (semG0, semG1)
    semS = (semS0, semS1)

    def _zero_row(r, _):
        for j in range(D // 16):
            zbuf_v[r, pl.ds(j * 16, 16)] = jnp.zeros((16,), jnp.float32)
        return 0
    lax.fori_loop(0, ZROWS, _zero_row, 0)
    for z in range(ROWS_PER_TILE // ZROWS):
        pltpu.sync_copy(zbuf_v, acc_sh.at[pl.ds(s * ROWS_PER_TILE + z * ZROWS, ZROWS)])
    plsc.subcore_barrier()

    base2 = (c * NS + s) * (EPW // 2)

    def _startA(i, q):
        off = base + i * K
        off2 = base2 + i * (K // 2)
        pltpu.async_copy(sidx_hbm.at[pl.ds(off, K)], sidx[q], semA[q])
        pltpu.async_copy(ridx_hbm.at[pl.ds(off, K)], ridx[q], semA[q])
        pltpu.async_copy(tpw_hbm.at[pl.ds(off2, K // 2)], tpwb[q], semA[q])

    def _waitA(i, q):
        off = base + i * K
        off2 = base2 + i * (K // 2)
        pltpu.make_async_copy(sidx_hbm.at[pl.ds(off, K)], sidx[q], semA[q]).wait()
        pltpu.make_async_copy(ridx_hbm.at[pl.ds(off, K)], ridx[q], semA[q]).wait()
        pltpu.make_async_copy(tpw_hbm.at[pl.ds(off2, K // 2)], tpwb[q], semA[q]).wait()

    def _startG(q, p):
        pltpu.async_copy(x_hbm.at[sidx[q]], xr[p], semG[p])

    def _waitG(q, p):
        pltpu.make_async_copy(x_hbm.at[sidx[q]], xr[p], semG[p]).wait()

    def _startS(q, p):
        pltpu.async_copy(xr[p], acc_sh.at[ridx[q]], semS[p], add=True)

    def _waitS(q, p):
        pltpu.make_async_copy(xr[p], acc_sh.at[ridx[q]], semS[p]).wait()

    _MSK = jnp.uint32(0xFFFF0000)
    _S16 = jnp.uint32(16)

    def _mul(q, p):
        tpw, x = tpwb[q], xr[p]

        def _m(m, _):
            for half in range(2):
                e = m + half * (K // 2)
                for j in range(D // 32):
                    ut = tpw[m, pl.ds(half * (D // 2) + j * 16, 16)]
                    ft = jax.lax.bitcast_convert_type(ut & _MSK, jnp.float32)
                    sl1 = pl.ds(j * 16, 16)
                    x[e, sl1] = ft * x[e, sl1]
                    gt = jax.lax.bitcast_convert_type(ut << _S16, jnp.float32)
                    sl2 = pl.ds(D // 2 + j * 16, 16)
                    x[e, sl2] = gt * x[e, sl2]
            return 0
        lax.fori_loop(0, K // 2, _m, 0)

    def _step(i, q, p, waitS_prev=True, gather_next=True, startA_next=True):
        _waitG(q, p)
        _mul(q, p)
        _startS(q, p)
        if gather_next:
            _waitA(i + 1, (q + 1) % 4)
        if waitS_prev:
            _waitS((q + 3) % 4, 1 - p)
        if gather_next:
            _startG((q + 1) % 4, 1 - p)
        if startA_next:
            _startA(i + 3, (q + 3) % 4)

    _startA(0, 0)
    _startA(1, 1)
    _startA(2, 2)
    _waitA(0, 0)
    _startG(0, 0)
    _step(0, 0, 0, waitS_prev=False)

    def _body(t, _):
        i0 = 4 * t + 1
        _step(i0, 1, 1)
        _step(i0 + 1, 2, 0)
        _step(i0 + 2, 3, 1)
        _step(i0 + 3, 0, 0)
        return 0
    lax.fori_loop(0, (NCHUNK - 5) // 4, _body, 0)

    _step(NCHUNK - 4, 1, 1)
    _step(NCHUNK - 3, 2, 0, startA_next=False)
    _step(NCHUNK - 2, 3, 1, startA_next=False)
    _step(NCHUNK - 1, 0, 0, gather_next=False, startA_next=False)
    _waitS(0, 0)

    plsc.subcore_barrier()
    r0 = s * ROWS_PER_TILE
    pltpu.sync_copy(acc_sh.at[pl.ds(r0, ROWS_PER_TILE)],
                    out_hbm.at[c, pl.ds(r0, ROWS_PER_TILE)])


_sc_scatter = pl.kernel(
    _sc_body,
    mesh=plsc.VectorSubcoreMesh(core_axis_name="c", subcore_axis_name="s"),
    out_type=jax.ShapeDtypeStruct((NC, NP, D), jnp.float32),
    scratch_types=(
        [pltpu.VMEM((K,), jnp.int32),
         pltpu.VMEM((K,), jnp.int32),
         pltpu.VMEM((K // 2, D), jnp.uint32)] * 4 + [
        pltpu.VMEM((K, D), jnp.float32),
        pltpu.VMEM((K, D), jnp.float32),
        pltpu.VMEM((ZROWS, D), jnp.float32),
        pltpu.VMEM_SHARED((NP, D), jnp.float32),
    ] + [pltpu.SemaphoreType.DMA] * 8),
)


def kernel(node_feats, edge_attrs, edge_feats, edge_index, W_up, W1, W2, W3, W4, W_lin):
    sender = edge_index[0]
    receiver = edge_index[1]

    x = pl.pallas_call(
        _xup_body,
        out_shape=jax.ShapeDtypeStruct((N, D), jnp.float32),
        grid=(N // _BN,),
        in_specs=[
            pl.BlockSpec((_BN, D), lambda i: (i, 0)),
            pl.BlockSpec((D, D), lambda i: (0, 0)),
        ],
        out_specs=pl.BlockSpec((_BN, D), lambda i: (i, 0)),
    )(node_feats, W_up)

    accs = []
    for seg in range(NSEG):
        lo = seg * ESEG
        tpw2_s = pl.pallas_call(
            _mlp_body,
            out_shape=jax.ShapeDtypeStruct((ESEG // 2, D), jnp.uint32),
            grid=(ESEG // _BE,),
            in_specs=[
                pl.BlockSpec((_BE, DE), lambda i: (i, 0)),
                pl.BlockSpec((_BE, 1), lambda i: (i, 0)),
                pl.BlockSpec((DE, H), lambda i: (0, 0)),
                pl.BlockSpec((H, H), lambda i: (0, 0)),
                pl.BlockSpec((H, H), lambda i: (0, 0)),
                pl.BlockSpec((H, D), lambda i: (0, 0)),
            ],
            out_specs=pl.BlockSpec((_BE // 2, D), lambda i: (i, 0)),
        )(edge_feats[lo:lo + ESEG], edge_attrs[lo:lo + ESEG], W1, W2, W3, W4)

        accs.append(_sc_scatter(x, tpw2_s,
                                sender[lo:lo + ESEG], receiver[lo:lo + ESEG]))

    out = pl.pallas_call(
        _final_body,
        out_shape=jax.ShapeDtypeStruct((N, D), jnp.float32),
        grid=(N // _BN,),
        in_specs=[pl.BlockSpec((NC, _BN, D), lambda i: (0, i, 0))] * NSEG + [
            pl.BlockSpec((D, D), lambda i: (0, 0)),
        ],
        out_specs=pl.BlockSpec((_BN, D), lambda i: (i, 0)),
    )(*accs, W_lin)

    return out.reshape(N, D, 1)
